# baseline (speedup 1.0000x reference)
import functools

import numpy as np
import jax
import jax.numpy as jnp
from jax.experimental import pallas as pl
from jax.experimental.pallas import tpu as pltpu
from jax.experimental.pallas import tpu_sc as plsc

D = 300
PD = 384
NEG = 0.01
_SCALE = float(1.0 / np.sqrt(float(D)))

_NC, _NS = 2, 16
_NW = _NC * _NS
_GCHUNK = 128


def _lrelu(x):
    return jnp.where(x >= 0, x, NEG * x)


def _mm(a, b):
    return jax.lax.dot_general(a, b, (((1,), (0,)), ((), ())),
                               preferred_element_type=jnp.float32)


def _mm_t(a, b):
    return jax.lax.dot_general(a, b, (((1,), (1,)), ((), ())),
                               preferred_element_type=jnp.float32)


def _proj_body(x1_ref, x2_ref, w1_ref, b1_ref, w2_ref, b2_ref, g_ref):
    n = x1_ref.shape[0]
    for gi, (x_ref, w_ref, b_ref) in enumerate(
            ((x1_ref, w1_ref, b1_ref), (x2_ref, w2_ref, b2_ref))):
        x = x_ref[...]
        for j in range(4):
            r = _mm(x, w_ref[j]) + b_ref[j:j + 1, :]
            base = (gi * 4 + j) * n
            g_ref[base:base + n, :] = r


def _e_body(ea1_ref, we1_ref, be1_ref, ea2_ref, we2_ref, be2_ref,
            e1_ref, e2_ref):
    e1_ref[...] = _mm(ea1_ref[...], we1_ref[...]) + be1_ref[...]
    e2_ref[...] = _mm(ea2_ref[...], we2_ref[...]) + be2_ref[...]


def _sc_gather(table, idx, nrows):
    b_per_w = nrows // _NW
    nch = b_per_w // _GCHUNK
    mesh = plsc.VectorSubcoreMesh(core_axis_name="c", subcore_axis_name="s")

    @functools.partial(
        pl.kernel, mesh=mesh,
        out_type=jax.ShapeDtypeStruct((nrows, PD), jnp.float32),
        scratch_types=[pltpu.VMEM((_GCHUNK,), jnp.int32),
                       pltpu.VMEM((_GCHUNK, PD), jnp.float32),
                       pltpu.SemaphoreType.DMA],
    )
    def knl(table_hbm, idx_hbm, out_hbm, idx_v, rows_v, sem):
        wid = jax.lax.axis_index("s") * _NC + jax.lax.axis_index("c")

        @pl.loop(0, nch)
        def _(j):
            base = (wid * nch + j) * _GCHUNK
            pltpu.sync_copy(idx_hbm.at[pl.ds(base, _GCHUNK)], idx_v)
            pltpu.async_copy(table_hbm.at[idx_v], rows_v, sem).wait()
            pltpu.sync_copy(rows_v, out_hbm.at[pl.ds(base, _GCHUNK)])

    return knl(table, idx)


def _fin_body(gath_ref, e_ref, s_ref, dst_ref, o_ref):
    n = o_ref.shape[0]
    ne = e_ref.shape[0]
    qd = gath_ref[0:ne, :]
    kj = gath_ref[ne:2 * ne, :]
    vj = gath_ref[2 * ne:3 * ne, :]
    e = e_ref[...]
    kje = kj + e
    vje = vj + e

    t = qd * kje
    alpha = _mm_t(jnp.ones((1, PD), jnp.float32), t) * _SCALE

    dst = dst_ref[...]
    row_ids = jax.lax.broadcasted_iota(jnp.int32, (n, ne), 0)
    mdst_b = (row_ids == dst)

    a_dense = jnp.where(mdst_b, alpha, -jnp.inf)
    amax = jnp.max(a_dense, axis=1, keepdims=True)
    amax = jnp.where(amax == -jnp.inf, 0.0, amax)
    p = jnp.exp(a_dense - amax)
    denom = jnp.sum(p, axis=1, keepdims=True)
    pn = p / (denom + 1e-16)

    out = _mm(pn, vje) + s_ref[...]
    o_ref[...] = _lrelu(out)


def _cross_body(x1_ref, x2_ref,
                wt_ref, bt_ref, wet_ref, bet_ref,
                wg_ref, bg_ref, weg_ref, beg_ref,
                w1_ref, b1_ref, w2_ref, b2_ref, w3_ref, b3_ref, o_ref):
    x1 = x1_ref[...]
    x2 = x2_ref[...]

    def conv(xd, xs, w, b, we, be):
        ec = jnp.sum(we[...], axis=0, keepdims=True) + be[...]
        qd = _mm(xd, w[0]) + b[0:1, :]
        ks = _mm(xs, w[1]) + b[1:2, :] + ec
        vs = _mm(xs, w[2]) + b[2:3, :] + ec
        sd = _mm(xd, w[3]) + b[3:4, :]
        al = _mm_t(qd, ks) * _SCALE
        amax = jnp.max(al, axis=1, keepdims=True)
        p = jnp.exp(al - amax)
        denom = jnp.sum(p, axis=1, keepdims=True)
        pn = p / (denom + 1e-16)
        return _lrelu(_mm(pn, vs) + sd)

    x1n = conv(x1, x2, wt_ref, bt_ref, wet_ref, bet_ref)
    x2n = conv(x2, x1, wg_ref, bg_ref, weg_ref, beg_ref)

    p1 = jnp.mean(x1n, axis=0, keepdims=True)
    p2 = jnp.mean(x2n, axis=0, keepdims=True)
    xc = jnp.concatenate([p1, p2], axis=1)
    h = _lrelu(_mm(xc, w1_ref[...]) + b1_ref[...])
    h = _lrelu(_mm(h, w2_ref[...]) + b2_ref[...])
    o = _mm(h, w3_ref[...]) + b3_ref[...]
    o = 1.0 / (1.0 + jnp.exp(-o))

    o_ref[...] = jnp.zeros((8, 2 * PD), jnp.float32)
    o_ref[0:1, 0:PD] = p1
    o_ref[1:2, 0:PD] = p2
    o_ref[2:3, 0:128] = o


def _padw(w):
    return jnp.pad(w, ((0, PD - D), (0, PD - D)))


def _padb(b):
    return jnp.pad(b, (0, PD - D)).reshape(1, PD)


def _stack_conv(p):
    w = jnp.stack([_padw(p['Wq'][0]), _padw(p['Wk'][0]),
                   _padw(p['Wv'][0]), _padw(p['Ws'][0])])
    b = jnp.concatenate([_padb(p['bq'][0]), _padb(p['bk'][0]),
                         _padb(p['bv'][0]), _padb(p['bs'][0])], axis=0)
    return w, b, _padw(p['We'][0]), _padb(p['be'][0])


def kernel(x_1, x_2, edge_idx_1, edge_idx_2, edge_attr_1, edge_attr_2, params):
    n = x_1.shape[0]
    ne = edge_idx_1.shape[1]
    f32 = jnp.float32

    padx = lambda x: jnp.pad(x, ((0, 0), (0, PD - D)))
    wT, bT, weT, beT = _stack_conv(params['TSA'])
    wG, bG, weG, beG = _stack_conv(params['GSA'])
    wTC, bTC, weTC, beTC = _stack_conv(params['TCA'])
    wGC, bGC, weGC, beGC = _stack_conv(params['GCA'])

    src1 = edge_idx_1[0].astype(jnp.int32)
    dst1 = edge_idx_1[1].astype(jnp.int32)
    src2 = edge_idx_2[0].astype(jnp.int32)
    dst2 = edge_idx_2[1].astype(jnp.int32)

    gtab = pl.pallas_call(
        _proj_body, out_shape=jax.ShapeDtypeStruct((8 * n, PD), f32))(
        padx(x_1), padx(x_2), wT, bT, wG, bG)

    e1, e2 = pl.pallas_call(
        _e_body, out_shape=[jax.ShapeDtypeStruct((ne, PD), f32)] * 2)(
        padx(edge_attr_1), weT, beT, padx(edge_attr_2), weG, beG)

    idx = jnp.concatenate([dst1, src1 + n, src1 + 2 * n,
                           dst2 + 4 * n, src2 + 5 * n, src2 + 6 * n])
    gath = _sc_gather(gtab, idx, 6 * ne)

    fin = [
        pl.pallas_call(
            _fin_body,
            grid=(1,),
            in_specs=[
                pl.BlockSpec((3 * ne, PD), lambda i, gi=gi: (gi, 0)),
                pl.BlockSpec((ne, PD), lambda i: (0, 0)),
                pl.BlockSpec((n, PD), lambda i, gi=gi: (3 + 4 * gi, 0)),
                pl.BlockSpec((1, ne), lambda i: (0, 0)),
            ],
            out_specs=pl.BlockSpec((n, PD), lambda i: (0, 0)),
            out_shape=jax.ShapeDtypeStruct((n, PD), f32))
        for gi in range(2)
    ]
    x1p = fin[0](gath, e1, gtab, dst1.reshape(1, ne))
    x2p = fin[1](gath, e2, gtab, dst2.reshape(1, ne))

    m = params['mlp']
    w1p = jnp.concatenate([jnp.pad(m['W1'][:D], ((0, PD - D), (0, 0))),
                           jnp.pad(m['W1'][D:], ((0, PD - D), (0, 0)))], axis=0)
    b1p = m['b1'].reshape(1, -1)
    b2p = m['b2'].reshape(1, -1)
    w3p = jnp.pad(m['W3'], ((0, 0), (0, 127)))
    b3p = jnp.pad(m['b3'], (0, 127)).reshape(1, 128)

    packed = pl.pallas_call(
        _cross_body, out_shape=jax.ShapeDtypeStruct((8, 2 * PD), f32))(
        x1p, x2p, wTC, bTC, weTC, beTC, wGC, bGC, weGC, beGC,
        w1p, b1p, m['W2'], b2p, w3p, b3p)

    p1 = packed[0, :D]
    p2 = packed[1, :D]
    out = packed[2, :1]
    return (p1, p2, out)


# speedup vs baseline: 1.4379x; 1.4379x over previous
import functools

import numpy as np
import jax
import jax.numpy as jnp
from jax.experimental import pallas as pl
from jax.experimental.pallas import tpu as pltpu
from jax.experimental.pallas import tpu_sc as plsc

D = 300
PD = 384
NEG = 0.01
_SCALE = float(1.0 / np.sqrt(float(D)))

_NC, _NS = 2, 16
_NW = _NC * _NS
_GCHUNK = 128


def _lrelu(x):
    return jnp.where(x >= 0, x, NEG * x)


def _mm(a, b):
    return jax.lax.dot_general(a, b, (((1,), (0,)), ((), ())),
                               preferred_element_type=jnp.float32)


def _mm_t(a, b):
    return jax.lax.dot_general(a, b, (((1,), (1,)), ((), ())),
                               preferred_element_type=jnp.float32)


def _proj_body(x1_ref, x2_ref, w1_ref, b1_ref, w2_ref, b2_ref, g_ref):
    n = x1_ref.shape[0]
    g_ref[...] = jnp.zeros(g_ref.shape, jnp.float32)
    for gi, (x_ref, w_ref, b_ref) in enumerate(
            ((x1_ref, w1_ref, b1_ref), (x2_ref, w2_ref, b2_ref))):
        x = x_ref[...]
        for j in range(4):
            r = _mm(x, w_ref[j]) + b_ref[j:j + 1, :]
            base = (gi * 4 + j) * n
            g_ref[base:base + n, 0:D] = r


def _e_body(ea1_ref, we1_ref, be1_ref, ea2_ref, we2_ref, be2_ref,
            e1_ref, e2_ref):
    for ea_ref, we_ref, be_ref, e_ref in (
            (ea1_ref, we1_ref, be1_ref, e1_ref),
            (ea2_ref, we2_ref, be2_ref, e2_ref)):
        e_ref[...] = jnp.zeros(e_ref.shape, jnp.float32)
        e_ref[:, 0:D] = _mm(ea_ref[...], we_ref[...]) + be_ref[...]


def _sc_gather(table, idx, nrows):
    b_per_w = nrows // _NW
    nch = b_per_w // _GCHUNK
    mesh = plsc.VectorSubcoreMesh(core_axis_name="c", subcore_axis_name="s")

    @functools.partial(
        pl.kernel, mesh=mesh,
        out_type=jax.ShapeDtypeStruct((nrows, PD), jnp.float32),
        scratch_types=[pltpu.VMEM((_GCHUNK,), jnp.int32),
                       pltpu.VMEM((_GCHUNK, PD), jnp.float32),
                       pltpu.SemaphoreType.DMA],
    )
    def knl(table_hbm, idx_hbm, out_hbm, idx_v, rows_v, sem):
        wid = jax.lax.axis_index("s") * _NC + jax.lax.axis_index("c")

        @pl.loop(0, nch)
        def _(j):
            base = (wid * nch + j) * _GCHUNK
            pltpu.sync_copy(idx_hbm.at[pl.ds(base, _GCHUNK)], idx_v)
            pltpu.async_copy(table_hbm.at[idx_v], rows_v, sem).wait()
            pltpu.sync_copy(rows_v, out_hbm.at[pl.ds(base, _GCHUNK)])

    return knl(table, idx)


def _fin_body(qd_ref, q_ref, k_ref, v_ref, s_ref, e_ref,
              src_ref, dst_ref, o_ref):
    n = o_ref.shape[0]
    ne = e_ref.shape[0]
    e = e_ref[...]
    qd = qd_ref[...]

    src = src_ref[...]
    dst = dst_ref[...]
    row_ids = jax.lax.broadcasted_iota(jnp.int32, (n, ne), 0)
    msrc = (row_ids == src).astype(jnp.float32)
    mdst_b = (row_ids == dst)

    sqk = _mm_t(q_ref[...], k_ref[...])
    rows = _mm(sqk, msrc)
    alpha1 = jnp.sum(jnp.where(mdst_b, rows, 0.0), axis=0, keepdims=True)
    alpha2 = _mm_t(jnp.ones((1, PD), jnp.float32), qd * e)
    alpha = (alpha1 + alpha2) * _SCALE

    a_dense = jnp.where(mdst_b, alpha, -jnp.inf)
    amax = jnp.max(a_dense, axis=1, keepdims=True)
    amax = jnp.where(amax == -jnp.inf, 0.0, amax)
    p = jnp.exp(a_dense - amax)
    denom = jnp.sum(p, axis=1, keepdims=True)
    pn = p / (denom + 1e-16)

    c = _mm_t(pn, msrc)
    out = _mm(c, v_ref[...]) + _mm(pn, e) + s_ref[...]
    o_ref[...] = _lrelu(out)[:, 0:D]


def _cross_body(x1_ref, x2_ref, wt_ref, bt_ref, wg_ref, bg_ref,
                w1_ref, b1_ref, w2_ref, b2_ref, w3_ref, b3_ref, o_ref):
    x1 = x1_ref[...]
    x2 = x2_ref[...]

    def conv(xd, xs, w, b):
        ec = jnp.sum(w[4], axis=0, keepdims=True) + b[4:5, :]
        qd = _mm(xd, w[0]) + b[0:1, :]
        ks = _mm(xs, w[1]) + b[1:2, :] + ec
        vs = _mm(xs, w[2]) + b[2:3, :] + ec
        sd = _mm(xd, w[3]) + b[3:4, :]
        al = _mm_t(qd, ks) * _SCALE
        amax = jnp.max(al, axis=1, keepdims=True)
        p = jnp.exp(al - amax)
        denom = jnp.sum(p, axis=1, keepdims=True)
        pn = p / (denom + 1e-16)
        return _lrelu(_mm(pn, vs) + sd)

    x1n = conv(x1, x2, wt_ref, bt_ref)
    x2n = conv(x2, x1, wg_ref, bg_ref)

    p1 = jnp.mean(x1n, axis=0, keepdims=True)
    p2 = jnp.mean(x2n, axis=0, keepdims=True)
    xc = jnp.concatenate([p1, p2], axis=1)
    h = _lrelu(_mm(xc, w1_ref[...]) + b1_ref[...])
    h = _lrelu(_mm(h, w2_ref[...]) + b2_ref[...])
    o = _mm(h, w3_ref[...]) + b3_ref[...]
    o = 1.0 / (1.0 + jnp.exp(-o))

    o_ref[...] = jnp.zeros((8, 2 * D), jnp.float32)
    o_ref[0:1, 0:D] = p1
    o_ref[1:2, 0:D] = p2
    o_ref[2:3, 0:128] = jnp.broadcast_to(o, (1, 128))


def _conv_w(p):
    w = jnp.stack([p['Wq'][0], p['Wk'][0], p['Wv'][0], p['Ws'][0], p['We'][0]])
    b = jnp.stack([p['bq'][0], p['bk'][0], p['bv'][0], p['bs'][0], p['be'][0]])
    return w, b


def kernel(x_1, x_2, edge_idx_1, edge_idx_2, edge_attr_1, edge_attr_2, params):
    n = x_1.shape[0]
    ne = edge_idx_1.shape[1]
    f32 = jnp.float32

    wT, bT = _conv_w(params['TSA'])
    wG, bG = _conv_w(params['GSA'])
    wTC, bTC = _conv_w(params['TCA'])
    wGC, bGC = _conv_w(params['GCA'])

    src1 = edge_idx_1[0].astype(jnp.int32)
    dst1 = edge_idx_1[1].astype(jnp.int32)
    src2 = edge_idx_2[0].astype(jnp.int32)
    dst2 = edge_idx_2[1].astype(jnp.int32)

    gtab = pl.pallas_call(
        _proj_body, out_shape=jax.ShapeDtypeStruct((8 * n, PD), f32))(
        x_1, x_2, wT[:4], bT[:4], wG[:4], bG[:4])

    e1, e2 = pl.pallas_call(
        _e_body, out_shape=[jax.ShapeDtypeStruct((ne, PD), f32)] * 2)(
        edge_attr_1, wT[4], bT[4:5], edge_attr_2, wG[4], bG[4:5])

    idx = jnp.concatenate([dst1, dst2 + 4 * n])
    qd = _sc_gather(gtab, idx, 2 * ne)

    def fin(gi, e, src, dst):
        blk = lambda r: pl.BlockSpec((n, PD), lambda i, r=r: (r, 0))
        return pl.pallas_call(
            _fin_body,
            grid=(1,),
            in_specs=[
                pl.BlockSpec((ne, PD), lambda i, gi=gi: (gi, 0)),
                blk(4 * gi + 0), blk(4 * gi + 1),
                blk(4 * gi + 2), blk(4 * gi + 3),
                pl.BlockSpec((ne, PD), lambda i: (0, 0)),
                pl.BlockSpec((1, ne), lambda i: (0, 0)),
                pl.BlockSpec((1, ne), lambda i: (0, 0)),
            ],
            out_specs=pl.BlockSpec((n, D), lambda i: (0, 0)),
            out_shape=jax.ShapeDtypeStruct((n, D), f32),
        )(qd, gtab, gtab, gtab, gtab, e, src.reshape(1, ne), dst.reshape(1, ne))

    x1p = fin(0, e1, src1, dst1)
    x2p = fin(1, e2, src2, dst2)

    m = params['mlp']
    packed = pl.pallas_call(
        _cross_body, out_shape=jax.ShapeDtypeStruct((8, 2 * D), f32))(
        x1p, x2p, wTC, bTC, wGC, bGC,
        m['W1'], m['b1'].reshape(1, -1), m['W2'], m['b2'].reshape(1, -1),
        m['W3'], m['b3'].reshape(1, 1))

    p1 = packed[0, :D]
    p2 = packed[1, :D]
    out = packed[2, :1]
    return (p1, p2, out)


# speedup vs baseline: 1.4534x; 1.0108x over previous
import functools

import numpy as np
import jax
import jax.numpy as jnp
from jax.experimental import pallas as pl
from jax.experimental.pallas import tpu as pltpu
from jax.experimental.pallas import tpu_sc as plsc

D = 300
PD = 384
NEG = 0.01
_SCALE = float(1.0 / np.sqrt(float(D)))

_NC, _NS = 2, 16
_NW = _NC * _NS
_GCHUNK = 128


def _lrelu(x):
    return jnp.where(x >= 0, x, NEG * x)


def _mm(a, b):
    return jax.lax.dot_general(a, b, (((1,), (0,)), ((), ())),
                               preferred_element_type=jnp.float32)


def _mm_t(a, b):
    return jax.lax.dot_general(a, b, (((1,), (1,)), ((), ())),
                               preferred_element_type=jnp.float32)


def _pre_body(x1_ref, x2_ref, ea1_ref, ea2_ref,
              wq1, wk1, wv1, ws1, we1, bq1, bk1, bv1, bs1, be1,
              wq2, wk2, wv2, ws2, we2, bq2, bk2, bv2, bs2, be2,
              g_ref, e1_ref, e2_ref):
    n = x1_ref.shape[0]
    g_ref[...] = jnp.zeros(g_ref.shape, jnp.float32)
    for gi, (x_ref, ws, bs) in enumerate((
            (x1_ref, (wq1, wk1, wv1, ws1), (bq1, bk1, bv1, bs1)),
            (x2_ref, (wq2, wk2, wv2, ws2), (bq2, bk2, bv2, bs2)))):
        x = x_ref[...]
        for j in range(4):
            base = (gi * 4 + j) * n
            g_ref[base:base + n, 0:D] = _mm(x, ws[j][...]) + bs[j][...]
    for ea_ref, we, be, e_ref in ((ea1_ref, we1, be1, e1_ref),
                                  (ea2_ref, we2, be2, e2_ref)):
        e_ref[...] = jnp.zeros(e_ref.shape, jnp.float32)
        e_ref[:, 0:D] = _mm(ea_ref[...], we[...]) + be[...]


def _sc_gather(table, idx, nrows):
    b_per_w = nrows // _NW
    nch = b_per_w // _GCHUNK
    mesh = plsc.VectorSubcoreMesh(core_axis_name="c", subcore_axis_name="s")

    @functools.partial(
        pl.kernel, mesh=mesh,
        out_type=jax.ShapeDtypeStruct((nrows, PD), jnp.float32),
        scratch_types=[pltpu.VMEM((_GCHUNK,), jnp.int32),
                       pltpu.VMEM((_GCHUNK, PD), jnp.float32),
                       pltpu.SemaphoreType.DMA],
    )
    def knl(table_hbm, idx_hbm, out_hbm, idx_v, rows_v, sem):
        wid = jax.lax.axis_index("s") * _NC + jax.lax.axis_index("c")

        @pl.loop(0, nch)
        def _(j):
            base = (wid * nch + j) * _GCHUNK
            pltpu.sync_copy(idx_hbm.at[pl.ds(base, _GCHUNK)], idx_v)
            pltpu.async_copy(table_hbm.at[idx_v], rows_v, sem).wait()
            pltpu.sync_copy(rows_v, out_hbm.at[pl.ds(base, _GCHUNK)])

    return knl(table, idx)


def _fin_body(qd_ref, q_ref, k_ref, v_ref, s_ref, e_ref,
              src_ref, dst_ref, o_ref):
    n = o_ref.shape[0]
    ne = e_ref.shape[0]
    e = e_ref[...]
    qd = qd_ref[...]

    src = src_ref[...]
    dst = dst_ref[...]
    row_ids = jax.lax.broadcasted_iota(jnp.int32, (n, ne), 0)
    msrc = (row_ids == src).astype(jnp.float32)
    mdst_b = (row_ids == dst)

    sqk = _mm_t(q_ref[...], k_ref[...])
    rows = _mm(sqk, msrc)
    alpha1 = jnp.sum(jnp.where(mdst_b, rows, 0.0), axis=0, keepdims=True)
    alpha2 = _mm_t(jnp.ones((1, PD), jnp.float32), qd * e)
    alpha = (alpha1 + alpha2) * _SCALE

    a_dense = jnp.where(mdst_b, alpha, -jnp.inf)
    amax = jnp.max(a_dense, axis=1, keepdims=True)
    amax = jnp.where(amax == -jnp.inf, 0.0, amax)
    p = jnp.exp(a_dense - amax)
    denom = jnp.sum(p, axis=1, keepdims=True)
    pn = p / (denom + 1e-16)

    c = _mm_t(pn, msrc)
    out = _mm(c, v_ref[...]) + _mm(pn, e) + s_ref[...]
    o_ref[...] = _lrelu(out)[:, 0:D]


def _cross_body(x1_ref, x2_ref,
                wqt, wkt, wvt, wst, wet, bqt, bkt, bvt, bst, bet,
                wqg, wkg, wvg, wsg, weg, bqg, bkg, bvg, bsg, beg,
                w1_ref, b1_ref, w2_ref, b2_ref, w3_ref, b3_ref, o_ref):
    x1 = x1_ref[...]
    x2 = x2_ref[...]

    def conv(xd, xs, wq, wk, wv, ws, we, bq, bk, bv, bs, be):
        ec = jnp.sum(we[...], axis=0, keepdims=True) + be[...]
        qd = _mm(xd, wq[...]) + bq[...]
        ks = _mm(xs, wk[...]) + bk[...] + ec
        vs = _mm(xs, wv[...]) + bv[...] + ec
        sd = _mm(xd, ws[...]) + bs[...]
        al = _mm_t(qd, ks) * _SCALE
        amax = jnp.max(al, axis=1, keepdims=True)
        p = jnp.exp(al - amax)
        denom = jnp.sum(p, axis=1, keepdims=True)
        pn = p / (denom + 1e-16)
        return _lrelu(_mm(pn, vs) + sd)

    x1n = conv(x1, x2, wqt, wkt, wvt, wst, wet, bqt, bkt, bvt, bst, bet)
    x2n = conv(x2, x1, wqg, wkg, wvg, wsg, weg, bqg, bkg, bvg, bsg, beg)

    p1 = jnp.mean(x1n, axis=0, keepdims=True)
    p2 = jnp.mean(x2n, axis=0, keepdims=True)
    xc = jnp.concatenate([p1, p2], axis=1)
    h = _lrelu(_mm(xc, w1_ref[...]) + b1_ref[...])
    h = _lrelu(_mm(h, w2_ref[...]) + b2_ref[...])
    o = _mm(h, w3_ref[...]) + b3_ref[...]
    o = 1.0 / (1.0 + jnp.exp(-o))

    o_ref[...] = jnp.zeros((8, 2 * D), jnp.float32)
    o_ref[0:1, 0:D] = p1
    o_ref[1:2, 0:D] = p2
    o_ref[2:3, 0:128] = jnp.broadcast_to(o, (1, 128))


def _conv_args(p):
    ws = [p['W' + nm][0] for nm in ('q', 'k', 'v', 's', 'e')]
    bs = [p['b' + nm][0].reshape(1, D) for nm in ('q', 'k', 'v', 's', 'e')]
    return ws + bs


def kernel(x_1, x_2, edge_idx_1, edge_idx_2, edge_attr_1, edge_attr_2, params):
    n = x_1.shape[0]
    ne = edge_idx_1.shape[1]
    f32 = jnp.float32

    src1 = edge_idx_1[0].astype(jnp.int32)
    dst1 = edge_idx_1[1].astype(jnp.int32)
    src2 = edge_idx_2[0].astype(jnp.int32)
    dst2 = edge_idx_2[1].astype(jnp.int32)

    gtab, e1, e2 = pl.pallas_call(
        _pre_body,
        out_shape=[jax.ShapeDtypeStruct((8 * n, PD), f32),
                   jax.ShapeDtypeStruct((ne, PD), f32),
                   jax.ShapeDtypeStruct((ne, PD), f32)])(
        x_1, x_2, edge_attr_1, edge_attr_2,
        *_conv_args(params['TSA']), *_conv_args(params['GSA']))

    idx = jnp.concatenate([dst1, dst2 + 4 * n])
    qd = _sc_gather(gtab, idx, 2 * ne)

    def fin(gi, e, src, dst):
        blk = lambda r: pl.BlockSpec((n, PD), lambda i, r=r: (r, 0))
        return pl.pallas_call(
            _fin_body,
            grid=(1,),
            in_specs=[
                pl.BlockSpec((ne, PD), lambda i, gi=gi: (gi, 0)),
                blk(4 * gi + 0), blk(4 * gi + 1),
                blk(4 * gi + 2), blk(4 * gi + 3),
                pl.BlockSpec((ne, PD), lambda i: (0, 0)),
                pl.BlockSpec((1, ne), lambda i: (0, 0)),
                pl.BlockSpec((1, ne), lambda i: (0, 0)),
            ],
            out_specs=pl.BlockSpec((n, D), lambda i: (0, 0)),
            out_shape=jax.ShapeDtypeStruct((n, D), f32),
        )(qd, gtab, gtab, gtab, gtab, e, src.reshape(1, ne), dst.reshape(1, ne))

    x1p = fin(0, e1, src1, dst1)
    x2p = fin(1, e2, src2, dst2)

    m = params['mlp']
    packed = pl.pallas_call(
        _cross_body, out_shape=jax.ShapeDtypeStruct((8, 2 * D), f32))(
        x1p, x2p,
        *_conv_args(params['TCA']), *_conv_args(params['GCA']),
        m['W1'], m['b1'].reshape(1, -1), m['W2'], m['b2'].reshape(1, -1),
        m['W3'], m['b3'].reshape(1, 1))

    p1 = packed[0, :D]
    p2 = packed[1, :D]
    out = packed[2, :1]
    return (p1, p2, out)


# speedup vs baseline: 1.5021x; 1.0335x over previous
import functools

import numpy as np
import jax
import jax.numpy as jnp
from jax.experimental import pallas as pl
from jax.experimental.pallas import tpu as pltpu
from jax.experimental.pallas import tpu_sc as plsc

D = 300
PD = 384
NEG = 0.01
_SCALE = float(1.0 / np.sqrt(float(D)))

_NC, _NS = 2, 16
_NW = _NC * _NS
_GCHUNK = 128


def _lrelu(x):
    return jnp.where(x >= 0, x, NEG * x)


def _mm(a, b):
    return jax.lax.dot_general(a, b, (((1,), (0,)), ((), ())),
                               preferred_element_type=jnp.float32)


def _mm_t(a, b):
    return jax.lax.dot_general(a, b, (((1,), (1,)), ((), ())),
                               preferred_element_type=jnp.float32)


def _pre_body(x1_ref, x2_ref, ea1_ref, ea2_ref, ei1_ref, ei2_ref,
              wq1, wk1, wv1, ws1, we1, bq1, bk1, bv1, bs1, be1,
              wq2, wk2, wv2, ws2, we2, bq2, bk2, bv2, bs2, be2,
              g_ref, e1_ref, e2_ref, idx_ref):
    n = x1_ref.shape[0]
    g_ref[...] = jnp.zeros(g_ref.shape, jnp.float32)
    for gi, (x_ref, ws, bs) in enumerate((
            (x1_ref, (wq1, wk1, wv1, ws1), (bq1, bk1, bv1, bs1)),
            (x2_ref, (wq2, wk2, wv2, ws2), (bq2, bk2, bv2, bs2)))):
        x = x_ref[...]
        for j in range(4):
            base = (gi * 4 + j) * n
            g_ref[base:base + n, 0:D] = _mm(x, ws[j][...]) + bs[j][...]
    for ea_ref, we, be, e_ref in ((ea1_ref, we1, be1, e1_ref),
                                  (ea2_ref, we2, be2, e2_ref)):
        e_ref[...] = jnp.zeros(e_ref.shape, jnp.float32)
        e_ref[:, 0:D] = _mm(ea_ref[...], we[...]) + be[...]
    idx_ref[0:1, :] = ei1_ref[1:2, :]
    idx_ref[1:2, :] = ei2_ref[1:2, :] + 4 * n


def _sc_gather(table, idx2d, g, ne):
    nch = ne // (_NW * _GCHUNK)
    mesh = plsc.VectorSubcoreMesh(core_axis_name="c", subcore_axis_name="s")

    @functools.partial(
        pl.kernel, mesh=mesh,
        out_type=jax.ShapeDtypeStruct((ne, PD), jnp.float32),
        scratch_types=[pltpu.VMEM((_GCHUNK,), jnp.int32),
                       pltpu.VMEM((_GCHUNK, PD), jnp.float32),
                       pltpu.SemaphoreType.DMA],
    )
    def knl(table_hbm, idx_hbm, out_hbm, idx_v, rows_v, sem):
        wid = jax.lax.axis_index("s") * _NC + jax.lax.axis_index("c")

        @pl.loop(0, nch)
        def _(j):
            base = (wid * nch + j) * _GCHUNK
            pltpu.sync_copy(idx_hbm.at[g, pl.ds(base, _GCHUNK)], idx_v)
            pltpu.async_copy(table_hbm.at[idx_v], rows_v, sem).wait()
            pltpu.sync_copy(rows_v, out_hbm.at[pl.ds(base, _GCHUNK)])

    return knl(table, idx2d)


def _fin_body(qd_ref, q_ref, k_ref, v_ref, s_ref, e_ref, ei_ref, o_ref):
    n = o_ref.shape[0]
    ne = e_ref.shape[0]
    e = e_ref[...]
    qd = qd_ref[...]

    src = ei_ref[0:1, :]
    dst = ei_ref[1:2, :]
    row_ids = jax.lax.broadcasted_iota(jnp.int32, (n, ne), 0)
    msrc = (row_ids == src).astype(jnp.float32)
    mdst_b = (row_ids == dst)

    sqk = _mm_t(q_ref[...], k_ref[...])
    rows = _mm(sqk, msrc)
    alpha1 = jnp.sum(jnp.where(mdst_b, rows, 0.0), axis=0, keepdims=True)
    alpha2 = _mm_t(jnp.ones((1, PD), jnp.float32), qd * e)
    alpha = (alpha1 + alpha2) * _SCALE

    a_dense = jnp.where(mdst_b, alpha, -jnp.inf)
    amax = jnp.max(a_dense, axis=1, keepdims=True)
    amax = jnp.where(amax == -jnp.inf, 0.0, amax)
    p = jnp.exp(a_dense - amax)
    denom = jnp.sum(p, axis=1, keepdims=True)
    pn = p / (denom + 1e-16)

    c = _mm_t(pn, msrc)
    out = _mm(c, v_ref[...]) + _mm(pn, e) + s_ref[...]
    o_ref[...] = _lrelu(out)[:, 0:D]


def _cross_body(x1_ref, x2_ref,
                wqt, wkt, wvt, wst, wet, bqt, bkt, bvt, bst, bet,
                wqg, wkg, wvg, wsg, weg, bqg, bkg, bvg, bsg, beg,
                w1_ref, b1_ref, w2_ref, b2_ref, w3_ref, b3_ref, o_ref):
    x1 = x1_ref[...]
    x2 = x2_ref[...]

    def conv(xd, xs, wq, wk, wv, ws, we, bq, bk, bv, bs, be):
        ec = jnp.sum(we[...], axis=0, keepdims=True) + be[...]
        qd = _mm(xd, wq[...]) + bq[...]
        ks = _mm(xs, wk[...]) + bk[...] + ec
        vs = _mm(xs, wv[...]) + bv[...] + ec
        sd = _mm(xd, ws[...]) + bs[...]
        al = _mm_t(qd, ks) * _SCALE
        amax = jnp.max(al, axis=1, keepdims=True)
        p = jnp.exp(al - amax)
        denom = jnp.sum(p, axis=1, keepdims=True)
        pn = p / (denom + 1e-16)
        return _lrelu(_mm(pn, vs) + sd)

    x1n = conv(x1, x2, wqt, wkt, wvt, wst, wet, bqt, bkt, bvt, bst, bet)
    x2n = conv(x2, x1, wqg, wkg, wvg, wsg, weg, bqg, bkg, bvg, bsg, beg)

    p1 = jnp.mean(x1n, axis=0, keepdims=True)
    p2 = jnp.mean(x2n, axis=0, keepdims=True)
    xc = jnp.concatenate([p1, p2], axis=1)
    h = _lrelu(_mm(xc, w1_ref[...]) + b1_ref[...])
    h = _lrelu(_mm(h, w2_ref[...]) + b2_ref[...])
    o = _mm(h, w3_ref[...]) + b3_ref[...]
    o = 1.0 / (1.0 + jnp.exp(-o))

    o_ref[...] = jnp.zeros((8, 2 * D), jnp.float32)
    o_ref[0:1, 0:D] = p1
    o_ref[1:2, 0:D] = p2
    o_ref[2:3, 0:128] = jnp.broadcast_to(o, (1, 128))


def _conv_args(p):
    ws = [p['W' + nm][0] for nm in ('q', 'k', 'v', 's', 'e')]
    bs = [p['b' + nm][0].reshape(1, D) for nm in ('q', 'k', 'v', 's', 'e')]
    return ws + bs


def kernel(x_1, x_2, edge_idx_1, edge_idx_2, edge_attr_1, edge_attr_2, params):
    n = x_1.shape[0]
    ne = edge_idx_1.shape[1]
    f32 = jnp.float32

    ei1 = edge_idx_1.astype(jnp.int32)
    ei2 = edge_idx_2.astype(jnp.int32)

    gtab, e1, e2, idx2d = pl.pallas_call(
        _pre_body,
        out_shape=[jax.ShapeDtypeStruct((8 * n, PD), f32),
                   jax.ShapeDtypeStruct((ne, PD), f32),
                   jax.ShapeDtypeStruct((ne, PD), f32),
                   jax.ShapeDtypeStruct((2, ne), jnp.int32)])(
        x_1, x_2, edge_attr_1, edge_attr_2, ei1, ei2,
        *_conv_args(params['TSA']), *_conv_args(params['GSA']))

    qd1 = _sc_gather(gtab, idx2d, 0, ne)
    qd2 = _sc_gather(gtab, idx2d, 1, ne)

    def fin(gi, qd, e, ei):
        blk = lambda r: pl.BlockSpec((n, PD), lambda i, r=r: (r, 0))
        return pl.pallas_call(
            _fin_body,
            grid=(1,),
            in_specs=[
                pl.BlockSpec((ne, PD), lambda i: (0, 0)),
                blk(4 * gi + 0), blk(4 * gi + 1),
                blk(4 * gi + 2), blk(4 * gi + 3),
                pl.BlockSpec((ne, PD), lambda i: (0, 0)),
                pl.BlockSpec((2, ne), lambda i: (0, 0)),
            ],
            out_specs=pl.BlockSpec((n, D), lambda i: (0, 0)),
            out_shape=jax.ShapeDtypeStruct((n, D), f32),
        )(qd, gtab, gtab, gtab, gtab, e, ei)

    x1p = fin(0, qd1, e1, ei1)
    x2p = fin(1, qd2, e2, ei2)

    m = params['mlp']
    packed = pl.pallas_call(
        _cross_body, out_shape=jax.ShapeDtypeStruct((8, 2 * D), f32))(
        x1p, x2p,
        *_conv_args(params['TCA']), *_conv_args(params['GCA']),
        m['W1'], m['b1'].reshape(1, -1), m['W2'], m['b2'].reshape(1, -1),
        m['W3'], m['b3'].reshape(1, 1))

    p1 = packed[0, :D]
    p2 = packed[1, :D]
    out = packed[2, :1]
    return (p1, p2, out)


# speedup vs baseline: 1.5342x; 1.0214x over previous
import functools

import numpy as np
import jax
import jax.numpy as jnp
from jax.experimental import pallas as pl
from jax.experimental.pallas import tpu as pltpu
from jax.experimental.pallas import tpu_sc as plsc

D = 300
PD = 384
NEG = 0.01
_SCALE = float(1.0 / np.sqrt(float(D)))

_NC, _NS = 2, 16
_NW = _NC * _NS
_GCHUNK = 128

_BF = jnp.bfloat16


def _lrelu(x):
    return jnp.where(x >= 0, x, NEG * x)


def _bf(x):
    return x.astype(_BF)


def _mm(a, b):
    return jax.lax.dot_general(_bf(a), _bf(b), (((1,), (0,)), ((), ())),
                               preferred_element_type=jnp.float32)


def _mm_t(a, b):
    return jax.lax.dot_general(_bf(a), _bf(b), (((1,), (1,)), ((), ())),
                               preferred_element_type=jnp.float32)


def _pre_body(x1_ref, x2_ref, ea1_ref, ea2_ref, ei1_ref, ei2_ref,
              w_ref, b_ref, g_ref, e1_ref, e2_ref, idx_ref):
    n = x1_ref.shape[0]
    g_ref[...] = jnp.zeros(g_ref.shape, jnp.float32)
    for gi, x_ref in enumerate((x1_ref, x2_ref)):
        x = x_ref[...]
        for j in range(4):
            base = (gi * 4 + j) * n
            wj = 5 * gi + j
            g_ref[base:base + n, 0:D] = (
                _mm(x, w_ref[wj]) + b_ref[wj:wj + 1, :])
    for gi, (ea_ref, e_ref) in enumerate(((ea1_ref, e1_ref),
                                          (ea2_ref, e2_ref))):
        wj = 5 * gi + 4
        e_ref[...] = jnp.zeros(e_ref.shape, jnp.float32)
        e_ref[:, 0:D] = _mm(ea_ref[...], w_ref[wj]) + b_ref[wj:wj + 1, :]
    idx_ref[0:1, :] = ei1_ref[1:2, :]
    idx_ref[1:2, :] = ei2_ref[1:2, :] + 4 * n


def _sc_gather(table, idx2d, g, ne):
    nch = ne // (_NW * _GCHUNK)
    mesh = plsc.VectorSubcoreMesh(core_axis_name="c", subcore_axis_name="s")

    @functools.partial(
        pl.kernel, mesh=mesh,
        out_type=jax.ShapeDtypeStruct((ne, PD), jnp.float32),
        scratch_types=[pltpu.VMEM((_GCHUNK,), jnp.int32),
                       pltpu.VMEM((_GCHUNK, PD), jnp.float32),
                       pltpu.SemaphoreType.DMA],
    )
    def knl(table_hbm, idx_hbm, out_hbm, idx_v, rows_v, sem):
        wid = jax.lax.axis_index("s") * _NC + jax.lax.axis_index("c")

        @pl.loop(0, nch)
        def _(j):
            base = (wid * nch + j) * _GCHUNK
            pltpu.sync_copy(idx_hbm.at[g, pl.ds(base, _GCHUNK)], idx_v)
            pltpu.async_copy(table_hbm.at[idx_v], rows_v, sem).wait()
            pltpu.sync_copy(rows_v, out_hbm.at[pl.ds(base, _GCHUNK)])

    return knl(table, idx2d)


def _fin_body(qd_ref, q_ref, k_ref, v_ref, s_ref, e_ref, ei_ref, o_ref):
    n = o_ref.shape[0]
    ne = e_ref.shape[0]
    e = e_ref[...]
    qd = qd_ref[...]

    src = ei_ref[0:1, :]
    dst = ei_ref[1:2, :]
    row_ids = jax.lax.broadcasted_iota(jnp.int32, (n, ne), 0)
    msrc = (row_ids == src).astype(_BF)
    mdst_b = (row_ids == dst)

    sqk = _mm_t(q_ref[...], k_ref[...])
    rows = _mm(sqk, msrc)
    alpha1 = jnp.sum(jnp.where(mdst_b, rows, 0.0), axis=0, keepdims=True)
    alpha2 = _mm_t(jnp.ones((1, PD), jnp.float32), qd * e)
    alpha = (alpha1 + alpha2) * _SCALE

    a_dense = jnp.where(mdst_b, alpha, -jnp.inf)
    amax = jnp.max(a_dense, axis=1, keepdims=True)
    amax = jnp.where(amax == -jnp.inf, 0.0, amax)
    p = jnp.exp(a_dense - amax)
    denom = jnp.sum(p, axis=1, keepdims=True)
    pn = p / (denom + 1e-16)

    c = _mm_t(pn, msrc)
    out = _mm(c, v_ref[...]) + _mm(pn, e) + s_ref[...]
    o_ref[...] = _lrelu(out)[:, 0:D]


def _cross_body(x1_ref, x2_ref, w_ref, b_ref,
                w1_ref, b1_ref, w2_ref, b2_ref, w3_ref, b3_ref, o_ref):
    x1 = x1_ref[...]
    x2 = x2_ref[...]

    def conv(xd, xs, o):
        ec = (jnp.sum(w_ref[o + 4].astype(jnp.float32), axis=0, keepdims=True)
              + b_ref[o + 4:o + 5, :])
        qd = _mm(xd, w_ref[o + 0]) + b_ref[o + 0:o + 1, :]
        ks = _mm(xs, w_ref[o + 1]) + b_ref[o + 1:o + 2, :] + ec
        vs = _mm(xs, w_ref[o + 2]) + b_ref[o + 2:o + 3, :] + ec
        sd = _mm(xd, w_ref[o + 3]) + b_ref[o + 3:o + 4, :]
        al = _mm_t(qd, ks) * _SCALE
        amax = jnp.max(al, axis=1, keepdims=True)
        p = jnp.exp(al - amax)
        denom = jnp.sum(p, axis=1, keepdims=True)
        pn = p / (denom + 1e-16)
        return _lrelu(_mm(pn, vs) + sd)

    x1n = conv(x1, x2, 0)
    x2n = conv(x2, x1, 5)

    p1 = jnp.mean(x1n, axis=0, keepdims=True)
    p2 = jnp.mean(x2n, axis=0, keepdims=True)
    xc = jnp.concatenate([p1, p2], axis=1)
    h = _lrelu(_mm(xc, w1_ref[...]) + b1_ref[...])
    h = _lrelu(_mm(h, w2_ref[...]) + b2_ref[...])
    o = _mm(h, w3_ref[...]) + b3_ref[...]
    o = 1.0 / (1.0 + jnp.exp(-o))

    o_ref[...] = jnp.zeros((8, 2 * D), jnp.float32)
    o_ref[0:1, 0:D] = p1
    o_ref[1:2, 0:D] = p2
    o_ref[2:3, 0:128] = jnp.broadcast_to(o, (1, 128))


def _pack(pa, pb):
    names = ('q', 'k', 'v', 's', 'e')
    w = jnp.stack([p['W' + nm][0] for p in (pa, pb) for nm in names])
    b = jnp.stack([p['b' + nm][0] for p in (pa, pb) for nm in names])
    return w.astype(_BF), b


def kernel(x_1, x_2, edge_idx_1, edge_idx_2, edge_attr_1, edge_attr_2, params):
    n = x_1.shape[0]
    ne = edge_idx_1.shape[1]
    f32 = jnp.float32

    ei1 = edge_idx_1.astype(jnp.int32)
    ei2 = edge_idx_2.astype(jnp.int32)
    wI, bI = _pack(params['TSA'], params['GSA'])
    wC, bC = _pack(params['TCA'], params['GCA'])

    gtab, e1, e2, idx2d = pl.pallas_call(
        _pre_body,
        out_shape=[jax.ShapeDtypeStruct((8 * n, PD), f32),
                   jax.ShapeDtypeStruct((ne, PD), f32),
                   jax.ShapeDtypeStruct((ne, PD), f32),
                   jax.ShapeDtypeStruct((2, ne), jnp.int32)])(
        x_1, x_2, edge_attr_1, edge_attr_2, ei1, ei2, wI, bI)

    qd1 = _sc_gather(gtab, idx2d, 0, ne)
    qd2 = _sc_gather(gtab, idx2d, 1, ne)

    def fin(gi, qd, e, ei):
        blk = lambda r: pl.BlockSpec((n, PD), lambda i, r=r: (r, 0))
        return pl.pallas_call(
            _fin_body,
            grid=(1,),
            in_specs=[
                pl.BlockSpec((ne, PD), lambda i: (0, 0)),
                blk(4 * gi + 0), blk(4 * gi + 1),
                blk(4 * gi + 2), blk(4 * gi + 3),
                pl.BlockSpec((ne, PD), lambda i: (0, 0)),
                pl.BlockSpec((2, ne), lambda i: (0, 0)),
            ],
            out_specs=pl.BlockSpec((n, D), lambda i: (0, 0)),
            out_shape=jax.ShapeDtypeStruct((n, D), f32),
        )(qd, gtab, gtab, gtab, gtab, e, ei)

    x1p = fin(0, qd1, e1, ei1)
    x2p = fin(1, qd2, e2, ei2)

    m = params['mlp']
    packed = pl.pallas_call(
        _cross_body, out_shape=jax.ShapeDtypeStruct((8, 2 * D), f32))(
        x1p, x2p, wC, bC,
        m['W1'], m['b1'].reshape(1, -1), m['W2'], m['b2'].reshape(1, -1),
        m['W3'], m['b3'].reshape(1, 1))

    p1 = packed[0, :D]
    p2 = packed[1, :D]
    out = packed[2, :1]
    return (p1, p2, out)
